# speedup vs baseline: 2.2278x; 2.2278x over previous
import functools

import jax
import jax.numpy as jnp
from jax import lax
from jax.experimental import pallas as pl
from jax.experimental.pallas import tpu as pltpu
from jax.experimental.pallas import tpu_sc as plsc

B, P, G, H, W = 4, 20000, 200, 512, 512
PPAD = 20480
BLK = 2048
NP = PPAD // BLK
NWORK = 32
CHUNK = (B * PPAD) // NWORK
NGATH = CHUNK // 128
VPB = CHUNK // 16
MAGIC = jnp.float32(2.0 ** 23)


def _sc_gather_labels(xs, ys, masks_flat):
    mesh = plsc.VectorSubcoreMesh(core_axis_name="c", subcore_axis_name="s")

    @functools.partial(
        pl.kernel,
        mesh=mesh,
        out_type=jax.ShapeDtypeStruct((B * PPAD,), jnp.int32),
        scratch_types=[
            pltpu.VMEM((CHUNK,), jnp.float32),
            pltpu.VMEM((CHUNK,), jnp.float32),
            pltpu.VMEM((CHUNK,), jnp.int32),
            pltpu.VMEM((CHUNK,), jnp.int32),
            pltpu.SemaphoreType.DMA,
        ],
    )
    def sc_kernel(xs_hbm, ys_hbm, masks_hbm, out_hbm, xv, yv, idxv, labv, sem):
        nc = 2
        wid = lax.axis_index("s") * nc + lax.axis_index("c")
        base = wid * CHUNK
        batch = base // PPAD
        hoff = batch * (H * W)
        pltpu.sync_copy(xs_hbm.at[pl.ds(base, CHUNK)], xv)
        pltpu.sync_copy(ys_hbm.at[pl.ds(base, CHUNK)], yv)

        def body(i, carry):
            x16 = xv[pl.ds(i * 16, 16)]
            y16 = yv[pl.ds(i * 16, 16)]
            rx = (x16 + MAGIC) - MAGIC
            ry = (y16 + MAGIC) - MAGIC
            rx = jnp.minimum(jnp.maximum(rx, 0.0), float(W - 1))
            ry = jnp.minimum(jnp.maximum(ry, 0.0), float(H - 1))
            xi = rx.astype(jnp.int32)
            yi = ry.astype(jnp.int32)
            idxv[pl.ds(i * 16, 16)] = yi * W + xi + hoff
            return carry

        lax.fori_loop(0, VPB, body, 0)

        copies = []
        for c in range(NGATH):
            copies.append(
                pltpu.async_copy(
                    masks_hbm.at[idxv.at[pl.ds(c * 128, 128)]],
                    labv.at[pl.ds(c * 128, 128)],
                    sem,
                )
            )
        for cp in copies:
            cp.wait()
        pltpu.sync_copy(labv, out_hbm.at[pl.ds(base, CHUNK)])

    return sc_kernel(xs, ys, masks_flat)


def _tc_match_kernel(pts_ref, tgt_ref, lab_ref, src_ref, cost_ref,
                     gmin, gidx, imin, iidx):
    ip = pl.program_id(1)
    inf = jnp.float32(jnp.inf)
    bigi = jnp.int32(2 ** 30)

    @pl.when(ip == 0)
    def _init():
        gmin[...] = jnp.full((G, 1), inf, jnp.float32)
        imin[...] = jnp.full((G, 1), inf, jnp.float32)
        gidx[...] = jnp.zeros((G, 1), jnp.int32)
        iidx[...] = jnp.zeros((G, 1), jnp.int32)

    ux = pts_ref[0, 0:1, :]
    uy = pts_ref[0, 1:2, :]
    vx = tgt_ref[0, :, 0:1]
    vy = tgt_ref[0, :, 1:2]
    dx = ux - vx
    dy = uy - vy
    s = jnp.sqrt(dx * dx + dy * dy + jnp.float32(1e-12))

    lab = lab_ref[0, :, :]
    ids = lax.broadcasted_iota(jnp.int32, (G, 1), 0) + 1
    inside = lab == ids
    pid = ip * BLK + lax.broadcasted_iota(jnp.int32, (1, BLK), 1)
    valid = pid < P
    pidb = jnp.broadcast_to(pid, (G, BLK))

    s_v = jnp.where(valid, s, inf)
    s_i = jnp.where(inside & valid, s, inf)

    bgmin = jnp.min(s_v, axis=1, keepdims=True)
    bgidx = jnp.min(jnp.where(s_v == bgmin, pidb, bigi), axis=1, keepdims=True)
    bimin = jnp.min(s_i, axis=1, keepdims=True)
    biidx = jnp.min(jnp.where(s_i == bimin, pidb, bigi), axis=1, keepdims=True)

    gidx[...] = jnp.where(bgmin < gmin[...], bgidx, gidx[...])
    gmin[...] = jnp.minimum(bgmin, gmin[...])
    iidx[...] = jnp.where(bimin < imin[...], biidx, iidx[...])
    imin[...] = jnp.minimum(bimin, imin[...])

    @pl.when(ip == NP - 1)
    def _fin():
        has = imin[...] < inf
        sel_min = jnp.where(has, imin[...], gmin[...])
        sel_idx = jnp.where(has, iidx[...], gidx[...])
        src_ref[0, :, :] = sel_idx
        cost_ref[0, :, :] = jnp.sum(sel_min, axis=0, keepdims=True)


def _tc_match(pts_t, tgt, labels3, interpret=False):
    return pl.pallas_call(
        _tc_match_kernel,
        grid=(B, NP),
        in_specs=[
            pl.BlockSpec((1, 2, BLK), lambda b, i: (b, 0, i)),
            pl.BlockSpec((1, G, 2), lambda b, i: (b, 0, 0)),
            pl.BlockSpec((1, 1, BLK), lambda b, i: (b, 0, i)),
        ],
        out_specs=[
            pl.BlockSpec((1, G, 1), lambda b, i: (b, 0, 0)),
            pl.BlockSpec((1, 1, 1), lambda b, i: (b, 0, 0)),
        ],
        out_shape=[
            jax.ShapeDtypeStruct((B, G, 1), jnp.int32),
            jax.ShapeDtypeStruct((B, 1, 1), jnp.float32),
        ],
        scratch_shapes=[
            pltpu.VMEM((G, 1), jnp.float32),
            pltpu.VMEM((G, 1), jnp.int32),
            pltpu.VMEM((G, 1), jnp.float32),
            pltpu.VMEM((G, 1), jnp.int32),
        ],
        interpret=interpret,
    )(pts_t, tgt, labels3)


def kernel(pred_points, target_points, target_masks):
    pad = PPAD - P
    xs = jnp.pad(pred_points[:, :, 0], ((0, 0), (0, pad))).reshape(-1)
    ys = jnp.pad(pred_points[:, :, 1], ((0, 0), (0, pad))).reshape(-1)
    masks_flat = target_masks.reshape(-1)

    labels = _sc_gather_labels(xs, ys, masks_flat)
    labels3 = labels.reshape(B, 1, PPAD)

    pts_t = jnp.pad(jnp.swapaxes(pred_points, 1, 2), ((0, 0), (0, 0), (0, pad)))
    src3, cost3 = _tc_match(pts_t, target_points, labels3)

    src = src3[:, :, 0]
    tgt = jnp.broadcast_to(jnp.arange(G, dtype=jnp.int32), (B, G))
    costs = cost3[:, 0, 0]
    return src, tgt, costs


# speedup vs baseline: 2.3606x; 1.0596x over previous
import functools

import jax
import jax.numpy as jnp
from jax import lax
from jax.experimental import pallas as pl
from jax.experimental.pallas import tpu as pltpu
from jax.experimental.pallas import tpu_sc as plsc

B, P, G, H, W = 4, 20000, 200, 512, 512
PPAD = 20480
BLK = 2048
NP = PPAD // BLK
NWORK = 32
CHUNK = (B * PPAD) // NWORK
NGATH = CHUNK // 128
VPB = CHUNK // 16
NPADTAIL = PPAD - P
MAGIC = jnp.float32(2.0 ** 23)


def _sc_gather_labels(xs, ys, masks_flat):
    mesh = plsc.VectorSubcoreMesh(core_axis_name="c", subcore_axis_name="s")

    @functools.partial(
        pl.kernel,
        mesh=mesh,
        out_type=jax.ShapeDtypeStruct((B * PPAD,), jnp.int32),
        scratch_types=[
            pltpu.VMEM((CHUNK,), jnp.float32),
            pltpu.VMEM((CHUNK,), jnp.float32),
            pltpu.VMEM((CHUNK,), jnp.int32),
            pltpu.VMEM((CHUNK,), jnp.int32),
            pltpu.SemaphoreType.DMA,
        ],
    )
    def sc_kernel(xs_hbm, ys_hbm, masks_hbm, out_hbm, xv, yv, idxv, labv, sem):
        nc = 2
        wid = lax.axis_index("s") * nc + lax.axis_index("c")
        base = wid * CHUNK
        batch = base // PPAD
        hoff = batch * (H * W)
        pltpu.sync_copy(xs_hbm.at[pl.ds(base, CHUNK)], xv)
        pltpu.sync_copy(ys_hbm.at[pl.ds(base, CHUNK)], yv)

        def body(i, carry):
            x16 = xv[pl.ds(i * 16, 16)]
            y16 = yv[pl.ds(i * 16, 16)]
            rx = (x16 + MAGIC) - MAGIC
            ry = (y16 + MAGIC) - MAGIC
            rx = jnp.minimum(jnp.maximum(rx, 0.0), float(W - 1))
            ry = jnp.minimum(jnp.maximum(ry, 0.0), float(H - 1))
            xi = rx.astype(jnp.int32)
            yi = ry.astype(jnp.int32)
            idxv[pl.ds(i * 16, 16)] = yi * W + xi + hoff
            return carry

        lax.fori_loop(0, VPB, body, 0)

        copies = []
        for c in range(NGATH):
            copies.append(
                pltpu.async_copy(
                    masks_hbm.at[idxv.at[pl.ds(c * 128, 128)]],
                    labv.at[pl.ds(c * 128, 128)],
                    sem,
                )
            )
        for cp in copies:
            cp.wait()

        @pl.when(wid % (PPAD // CHUNK) == (PPAD // CHUNK) - 1)
        def _zero_pad():
            def zbody(i, carry):
                labv[pl.ds((CHUNK - NPADTAIL) + i * 16, 16)] = jnp.zeros(
                    (16,), jnp.int32
                )
                return carry

            lax.fori_loop(0, NPADTAIL // 16, zbody, 0)

        pltpu.sync_copy(labv, out_hbm.at[pl.ds(base, CHUNK)])

    return sc_kernel(xs, ys, masks_flat)


def _tc_match_kernel(pts_ref, tgt_ref, lab_ref, src_ref, cost_ref,
                     gmin, gidx, imin, iidx):
    ip = pl.program_id(1)
    inf = jnp.float32(jnp.inf)
    bigi = jnp.int32(2 ** 30)

    @pl.when(ip == 0)
    def _init():
        gmin[...] = jnp.full((G, 1), inf, jnp.float32)
        imin[...] = jnp.full((G, 1), inf, jnp.float32)
        gidx[...] = jnp.zeros((G, 1), jnp.int32)
        iidx[...] = jnp.zeros((G, 1), jnp.int32)

    ux = pts_ref[0, 0:1, :]
    uy = pts_ref[0, 1:2, :]
    vx = tgt_ref[0, :, 0:1]
    vy = tgt_ref[0, :, 1:2]
    dx = ux - vx
    dy = uy - vy
    s = jnp.sqrt(dx * dx + dy * dy + jnp.float32(1e-12))

    lab = lab_ref[0, :, :]
    ids = lax.broadcasted_iota(jnp.int32, (G, 1), 0) + 1
    inside = lab == ids
    pid = ip * BLK + lax.broadcasted_iota(jnp.int32, (1, BLK), 1)
    pidb = jnp.broadcast_to(pid, (G, BLK))

    s_i = jnp.where(inside, s, inf)

    bgmin = jnp.min(s, axis=1, keepdims=True)
    bgidx = jnp.min(jnp.where(s == bgmin, pidb, bigi), axis=1, keepdims=True)
    bimin = jnp.min(s_i, axis=1, keepdims=True)
    biidx = jnp.min(jnp.where(s_i == bimin, pidb, bigi), axis=1, keepdims=True)

    gidx[...] = jnp.where(bgmin < gmin[...], bgidx, gidx[...])
    gmin[...] = jnp.minimum(bgmin, gmin[...])
    iidx[...] = jnp.where(bimin < imin[...], biidx, iidx[...])
    imin[...] = jnp.minimum(bimin, imin[...])

    @pl.when(ip == NP - 1)
    def _fin():
        has = imin[...] < inf
        sel_min = jnp.where(has, imin[...], gmin[...])
        sel_idx = jnp.where(has, iidx[...], gidx[...])
        src_ref[0, :, :] = sel_idx
        cost_ref[0, :, :] = jnp.sum(sel_min, axis=0, keepdims=True)


def _tc_match(pts_t, tgt, labels3, interpret=False):
    return pl.pallas_call(
        _tc_match_kernel,
        grid=(B, NP),
        in_specs=[
            pl.BlockSpec((1, 2, BLK), lambda b, i: (b, 0, i)),
            pl.BlockSpec((1, G, 2), lambda b, i: (b, 0, 0)),
            pl.BlockSpec((1, 1, BLK), lambda b, i: (b, 0, i)),
        ],
        out_specs=[
            pl.BlockSpec((1, G, 1), lambda b, i: (b, 0, 0)),
            pl.BlockSpec((1, 1, 1), lambda b, i: (b, 0, 0)),
        ],
        out_shape=[
            jax.ShapeDtypeStruct((B, G, 1), jnp.int32),
            jax.ShapeDtypeStruct((B, 1, 1), jnp.float32),
        ],
        scratch_shapes=[
            pltpu.VMEM((G, 1), jnp.float32),
            pltpu.VMEM((G, 1), jnp.int32),
            pltpu.VMEM((G, 1), jnp.float32),
            pltpu.VMEM((G, 1), jnp.int32),
        ],
        interpret=interpret,
    )(pts_t, tgt, labels3)


def kernel(pred_points, target_points, target_masks):
    pad = PPAD - P
    xs = jnp.pad(pred_points[:, :, 0], ((0, 0), (0, pad))).reshape(-1)
    ys = jnp.pad(pred_points[:, :, 1], ((0, 0), (0, pad))).reshape(-1)
    masks_flat = target_masks.reshape(-1)

    labels = _sc_gather_labels(xs, ys, masks_flat)
    labels3 = labels.reshape(B, 1, PPAD)

    pts_t = jnp.pad(
        jnp.swapaxes(pred_points, 1, 2),
        ((0, 0), (0, 0), (0, pad)),
        constant_values=1e6,
    )
    src3, cost3 = _tc_match(pts_t, target_points, labels3)

    src = src3[:, :, 0]
    tgt = jnp.broadcast_to(jnp.arange(G, dtype=jnp.int32), (B, G))
    costs = cost3[:, 0, 0]
    return src, tgt, costs


# speedup vs baseline: 2.6244x; 1.1118x over previous
import functools

import jax
import jax.numpy as jnp
from jax import lax
from jax.experimental import pallas as pl
from jax.experimental.pallas import tpu as pltpu
from jax.experimental.pallas import tpu_sc as plsc

B, P, G, H, W = 4, 20000, 200, 512, 512
PPAD = 20480
BLK = 4096
NP = PPAD // BLK
NWORK = 32
CHUNK = (B * PPAD) // NWORK
NGATH = CHUNK // 128
VPB = CHUNK // 16
NPADTAIL = PPAD - P
MAGIC = 2.0 ** 23


def _sc_gather_labels(xs, ys, masks_flat):
    mesh = plsc.VectorSubcoreMesh(core_axis_name="c", subcore_axis_name="s")

    @functools.partial(
        pl.kernel,
        mesh=mesh,
        out_type=jax.ShapeDtypeStruct((B * PPAD,), jnp.int32),
        scratch_types=[
            pltpu.VMEM((CHUNK,), jnp.float32),
            pltpu.VMEM((CHUNK,), jnp.float32),
            pltpu.VMEM((CHUNK,), jnp.int32),
            pltpu.VMEM((CHUNK,), jnp.int32),
            pltpu.SemaphoreType.DMA,
        ],
    )
    def sc_kernel(xs_hbm, ys_hbm, masks_hbm, out_hbm, xv, yv, idxv, labv, sem):
        nc = 2
        wid = lax.axis_index("s") * nc + lax.axis_index("c")
        base = wid * CHUNK
        batch = base // PPAD
        hoff = batch * (H * W)
        pltpu.sync_copy(xs_hbm.at[pl.ds(base, CHUNK)], xv)
        pltpu.sync_copy(ys_hbm.at[pl.ds(base, CHUNK)], yv)

        def body(i, carry):
            x16 = xv[pl.ds(i * 16, 16)]
            y16 = yv[pl.ds(i * 16, 16)]
            rx = (x16 + MAGIC) - MAGIC
            ry = (y16 + MAGIC) - MAGIC
            rx = jnp.minimum(jnp.maximum(rx, 0.0), float(W - 1))
            ry = jnp.minimum(jnp.maximum(ry, 0.0), float(H - 1))
            xi = rx.astype(jnp.int32)
            yi = ry.astype(jnp.int32)
            idxv[pl.ds(i * 16, 16)] = yi * W + xi + hoff
            return carry

        lax.fori_loop(0, VPB, body, 0)

        copies = []
        for c in range(NGATH):
            copies.append(
                pltpu.async_copy(
                    masks_hbm.at[idxv.at[pl.ds(c * 128, 128)]],
                    labv.at[pl.ds(c * 128, 128)],
                    sem,
                )
            )
        for cp in copies:
            cp.wait()

        @pl.when(wid % (PPAD // CHUNK) == (PPAD // CHUNK) - 1)
        def _zero_pad():
            def zbody(i, carry):
                labv[pl.ds((CHUNK - NPADTAIL) + i * 16, 16)] = jnp.zeros(
                    (16,), jnp.int32
                )
                return carry

            lax.fori_loop(0, NPADTAIL // 16, zbody, 0)

        pltpu.sync_copy(labv, out_hbm.at[pl.ds(base, CHUNK)])

    return sc_kernel(xs, ys, masks_flat)


def _tc_match_kernel(pts_ref, tgt_ref, lab_ref, src_ref, cost_ref,
                     gmin, gidx, imin, iidx):
    ip = pl.program_id(1)
    inf = jnp.float32(jnp.inf)
    bigf = jnp.float32(1e9)

    @pl.when(ip == 0)
    def _init():
        gmin[...] = jnp.full((G, 1), inf, jnp.float32)
        imin[...] = jnp.full((G, 1), inf, jnp.float32)
        gidx[...] = jnp.zeros((G, 1), jnp.float32)
        iidx[...] = jnp.zeros((G, 1), jnp.float32)

    ux = pts_ref[0, 0:1, :]
    uy = pts_ref[0, 1:2, :]
    vx = tgt_ref[0, :, 0:1]
    vy = tgt_ref[0, :, 1:2]
    dx = ux - vx
    dy = uy - vy
    s = jnp.sqrt(dx * dx + dy * dy + jnp.float32(1e-12))

    lab = lab_ref[0, :, :]
    ids = lax.broadcasted_iota(jnp.int32, (G, 1), 0) + 1
    inside = lab == ids
    pidf = jnp.float32(ip * BLK) + lax.broadcasted_iota(
        jnp.int32, (1, BLK), 1).astype(jnp.float32)
    pidb = jnp.broadcast_to(pidf, (G, BLK))

    s_i = jnp.where(inside, s, inf)

    bgmin = jnp.min(s, axis=1, keepdims=True)
    bgidx = jnp.min(jnp.where(s == bgmin, pidb, bigf), axis=1, keepdims=True)
    bimin = jnp.min(s_i, axis=1, keepdims=True)
    biidx = jnp.min(jnp.where(s_i == bimin, pidb, bigf), axis=1, keepdims=True)

    gidx[...] = jnp.where(bgmin < gmin[...], bgidx, gidx[...])
    gmin[...] = jnp.minimum(bgmin, gmin[...])
    iidx[...] = jnp.where(bimin < imin[...], biidx, iidx[...])
    imin[...] = jnp.minimum(bimin, imin[...])

    @pl.when(ip == NP - 1)
    def _fin():
        has = imin[...] < inf
        sel_min = jnp.where(has, imin[...], gmin[...])
        sel_idx = jnp.where(has, iidx[...], gidx[...])
        src_ref[0, :, :] = sel_idx.astype(jnp.int32)
        cost_ref[0, :, :] = jnp.sum(sel_min, axis=0, keepdims=True)


def _tc_match(pts_t, tgt, labels3, interpret=False):
    return pl.pallas_call(
        _tc_match_kernel,
        grid=(B, NP),
        in_specs=[
            pl.BlockSpec((1, 2, BLK), lambda b, i: (b, 0, i)),
            pl.BlockSpec((1, G, 2), lambda b, i: (b, 0, 0)),
            pl.BlockSpec((1, 1, BLK), lambda b, i: (b, 0, i)),
        ],
        out_specs=[
            pl.BlockSpec((1, G, 1), lambda b, i: (b, 0, 0)),
            pl.BlockSpec((1, 1, 1), lambda b, i: (b, 0, 0)),
        ],
        out_shape=[
            jax.ShapeDtypeStruct((B, G, 1), jnp.int32),
            jax.ShapeDtypeStruct((B, 1, 1), jnp.float32),
        ],
        scratch_shapes=[
            pltpu.VMEM((G, 1), jnp.float32),
            pltpu.VMEM((G, 1), jnp.float32),
            pltpu.VMEM((G, 1), jnp.float32),
            pltpu.VMEM((G, 1), jnp.float32),
        ],
        interpret=interpret,
    )(pts_t, tgt, labels3)


def kernel(pred_points, target_points, target_masks):
    pad = PPAD - P
    xs = jnp.pad(pred_points[:, :, 0], ((0, 0), (0, pad))).reshape(-1)
    ys = jnp.pad(pred_points[:, :, 1], ((0, 0), (0, pad))).reshape(-1)
    masks_flat = target_masks.reshape(-1)

    labels = _sc_gather_labels(xs, ys, masks_flat)
    labels3 = labels.reshape(B, 1, PPAD)

    pts_t = jnp.pad(
        jnp.swapaxes(pred_points, 1, 2),
        ((0, 0), (0, 0), (0, pad)),
        constant_values=1e6,
    )
    src3, cost3 = _tc_match(pts_t, target_points, labels3)

    src = src3[:, :, 0]
    tgt = jnp.broadcast_to(jnp.arange(G, dtype=jnp.int32), (B, G))
    costs = cost3[:, 0, 0]
    return src, tgt, costs
